# baseline (speedup 1.0000x reference)
import functools

import jax
import jax.numpy as jnp
from jax import lax
from jax.experimental import pallas as pl
from jax.experimental.pallas import tpu as pltpu
from jax.experimental.pallas import tpu_sc as plsc

_LP = 8


def kernel(idxs, locs, data):
    B, N, D = locs.shape
    C = data.shape[2]
    RT = B * N

    CHUNK = 128
    assert RT % CHUNK == 0
    NCHUNKS = RT // CHUNK
    G = 4
    GR = G * CHUNK

    info = plsc.get_sparse_core_info()
    NC, NS = info.num_cores, info.num_subcores
    NW = NC * NS
    NGROUPS = -(-NCHUNKS // (G * NW)) * NW
    ITERS = NGROUPS // NW
    RT_PAD = NGROUPS * GR

    idxs_flat = jnp.pad(idxs.reshape(RT), (0, RT_PAD - RT))
    locs2d = jnp.pad(locs.reshape(RT, D), ((0, 0), (0, _LP - D)))
    data2d = data.reshape(RT, C)

    mesh = plsc.VectorSubcoreMesh(core_axis_name="c", subcore_axis_name="s")

    @functools.partial(
        pl.kernel,
        mesh=mesh,
        out_type=[
            jax.ShapeDtypeStruct((RT, D), jnp.float32),
            jax.ShapeDtypeStruct((RT, C), jnp.float32),
        ],
        scratch_types=[
            pltpu.VMEM((GR,), jnp.int32),
            pltpu.VMEM((GR,), jnp.int32),
            pltpu.VMEM((GR, _LP), jnp.float32),
            pltpu.VMEM((GR, _LP), jnp.float32),
            pltpu.VMEM((GR, C), jnp.float32),
            pltpu.VMEM((GR, C), jnp.float32),
            pltpu.SemaphoreType.DMA,
            pltpu.SemaphoreType.DMA,
            pltpu.SemaphoreType.DMA,
            pltpu.SemaphoreType.DMA,
            pltpu.SemaphoreType.DMA,
            pltpu.SemaphoreType.DMA,
        ],
        compiler_params=pltpu.CompilerParams(use_tc_tiling_on_sc=False),
    )
    def k(idx_hbm, locs_hbm, data_hbm, outl_hbm, outd_hbm,
          ib0, ib1, lb0, lb1, db0, db1, si0, si1, sg0, sg1, sw0, sw1):
        wid = lax.axis_index("s") * NC + lax.axis_index("c")
        IB, LB, DB = (ib0, ib1), (lb0, lb1), (db0, db1)
        SI, SG, SW = (si0, si1), (sg0, sg1), (sw0, sw1)

        def grp(t):
            return (wid + NW * t) * GR

        def stage(t, u):
            s, o = u, 1 - u

            @pl.when(jnp.logical_and(t >= 2, t <= ITERS + 1))
            def _():
                g_row = grp(t - 2)
                for j in range(G):
                    rb = g_row + j * CHUNK

                    @pl.when(rb < RT)
                    def _():
                        pltpu.make_async_copy(
                            DB[s].at[pl.ds(j * CHUNK, CHUNK)],
                            outd_hbm.at[pl.ds(rb, CHUNK)], SW[s]).wait()
                        pltpu.make_async_copy(
                            LB[s].at[pl.ds(j * CHUNK, CHUNK), pl.ds(0, D)],
                            outl_hbm.at[pl.ds(rb, CHUNK)], SW[s]).wait()

            @pl.when(jnp.logical_and(t >= 1, t <= ITERS))
            def _():
                g_row = grp(t - 1)
                for j in range(G):
                    isl = IB[o].at[pl.ds(j * CHUNK, CHUNK)]
                    pltpu.make_async_copy(
                        data_hbm.at[isl],
                        DB[o].at[pl.ds(j * CHUNK, CHUNK)], SG[o]).wait()
                    pltpu.make_async_copy(
                        locs_hbm.at[isl],
                        LB[o].at[pl.ds(j * CHUNK, CHUNK)], SG[o]).wait()
                for j in range(G):
                    rb = g_row + j * CHUNK

                    @pl.when(rb < RT)
                    def _():
                        pltpu.async_copy(
                            DB[o].at[pl.ds(j * CHUNK, CHUNK)],
                            outd_hbm.at[pl.ds(rb, CHUNK)], SW[o])
                        pltpu.async_copy(
                            LB[o].at[pl.ds(j * CHUNK, CHUNK), pl.ds(0, D)],
                            outl_hbm.at[pl.ds(rb, CHUNK)], SW[o])

            @pl.when(t + 1 <= ITERS - 1)
            def _():
                pltpu.async_copy(idx_hbm.at[pl.ds(grp(t + 1), GR)], IB[o], SI[o])

            @pl.when(t <= ITERS - 1)
            def _():
                pltpu.make_async_copy(
                    idx_hbm.at[pl.ds(grp(t), GR)], IB[s], SI[s]).wait()
                g_row = grp(t)
                for i in range(GR // 16):
                    pos = g_row + i * 16 + lax.iota(jnp.int32, 16)
                    v = IB[s][pl.ds(i * 16, 16)] + lax.div(pos, N) * N
                    IB[s][pl.ds(i * 16, 16)] = jnp.minimum(v, RT - 1)
                for j in range(G):
                    isl = IB[s].at[pl.ds(j * CHUNK, CHUNK)]
                    pltpu.async_copy(
                        data_hbm.at[isl],
                        DB[s].at[pl.ds(j * CHUNK, CHUNK)], SG[s])
                    pltpu.async_copy(
                        locs_hbm.at[isl],
                        LB[s].at[pl.ds(j * CHUNK, CHUNK)], SG[s])

        pltpu.async_copy(idx_hbm.at[pl.ds(grp(0), GR)], IB[0], SI[0])

        def body(tt, carry):
            stage(2 * tt, 0)
            stage(2 * tt + 1, 1)
            return carry

        lax.fori_loop(0, (ITERS + 3) // 2, body, 0)

    outl, outd = k(idxs_flat, locs2d, data2d)
    return (outl.reshape(B, N, D), outd.reshape(B, N, C))


# speedup vs baseline: 1.0648x; 1.0648x over previous
import functools

import jax
import jax.numpy as jnp
from jax import lax
from jax.experimental import pallas as pl
from jax.experimental.pallas import tpu as pltpu
from jax.experimental.pallas import tpu_sc as plsc

_LP = 8


def kernel(idxs, locs, data):
    B, N, D = locs.shape
    C = data.shape[2]
    RT = B * N

    GR = 800
    assert RT % GR == 0
    NGROUPS = RT // GR

    info = plsc.get_sparse_core_info()
    NC, NS = info.num_cores, info.num_subcores
    NW = NC * NS
    ITERS = -(-NGROUPS // NW)

    idxs_flat = idxs.reshape(RT)
    locs2d = jnp.pad(locs.reshape(RT, D), ((0, 0), (0, _LP - D)))
    data2d = data.reshape(RT, C)

    mesh = plsc.VectorSubcoreMesh(core_axis_name="c", subcore_axis_name="s")

    @functools.partial(
        pl.kernel,
        mesh=mesh,
        out_type=[
            jax.ShapeDtypeStruct((RT, D), jnp.float32),
            jax.ShapeDtypeStruct((RT, C), jnp.float32),
        ],
        scratch_types=[
            pltpu.VMEM((GR,), jnp.int32),
            pltpu.VMEM((GR,), jnp.int32),
            pltpu.VMEM((GR, _LP), jnp.float32),
            pltpu.VMEM((GR, _LP), jnp.float32),
            pltpu.VMEM((GR, C), jnp.float32),
            pltpu.VMEM((GR, C), jnp.float32),
            pltpu.SemaphoreType.DMA,
            pltpu.SemaphoreType.DMA,
            pltpu.SemaphoreType.DMA,
            pltpu.SemaphoreType.DMA,
            pltpu.SemaphoreType.DMA,
            pltpu.SemaphoreType.DMA,
        ],
        compiler_params=pltpu.CompilerParams(use_tc_tiling_on_sc=False),
    )
    def k(idx_hbm, locs_hbm, data_hbm, outl_hbm, outd_hbm,
          ib0, ib1, lb0, lb1, db0, db1, si0, si1, sg0, sg1, sw0, sw1):
        wid = lax.axis_index("s") * NC + lax.axis_index("c")
        IB, LB, DB = (ib0, ib1), (lb0, lb1), (db0, db1)
        SI, SG, SW = (si0, si1), (sg0, sg1), (sw0, sw1)

        def gid(t):
            return wid + NW * t

        def stage(t, u):
            s, o = u, 1 - u

            @pl.when(jnp.logical_and(t >= 2, gid(t - 2) < NGROUPS))
            def _():
                g_row = gid(t - 2) * GR
                pltpu.make_async_copy(
                    DB[s], outd_hbm.at[pl.ds(g_row, GR)], SW[s]).wait()
                pltpu.make_async_copy(
                    LB[s].at[:, pl.ds(0, D)],
                    outl_hbm.at[pl.ds(g_row, GR)], SW[s]).wait()

            @pl.when(jnp.logical_and(
                jnp.logical_and(t >= 1, t <= ITERS), gid(t - 1) < NGROUPS))
            def _():
                g_row = gid(t - 1) * GR
                pltpu.make_async_copy(data_hbm.at[IB[o]], DB[o], SG[o]).wait()
                pltpu.make_async_copy(locs_hbm.at[IB[o]], LB[o], SG[o]).wait()
                pltpu.async_copy(DB[o], outd_hbm.at[pl.ds(g_row, GR)], SW[o])
                pltpu.async_copy(
                    LB[o].at[:, pl.ds(0, D)], outl_hbm.at[pl.ds(g_row, GR)], SW[o])

            @pl.when(jnp.logical_and(t + 1 <= ITERS - 1, gid(t + 1) < NGROUPS))
            def _():
                pltpu.async_copy(
                    idx_hbm.at[pl.ds(gid(t + 1) * GR, GR)], IB[o], SI[o])

            @pl.when(jnp.logical_and(t <= ITERS - 1, gid(t) < NGROUPS))
            def _():
                g_row = gid(t) * GR
                pltpu.make_async_copy(
                    idx_hbm.at[pl.ds(g_row, GR)], IB[s], SI[s]).wait()

                def conv(i, carry):
                    off = pl.multiple_of(i * 16, 16)
                    pos = g_row + off + lax.iota(jnp.int32, 16)
                    IB[s][pl.ds(off, 16)] = (
                        IB[s][pl.ds(off, 16)] + lax.div(pos, N) * N)
                    return carry

                lax.fori_loop(0, GR // 16, conv, 0)
                pltpu.async_copy(data_hbm.at[IB[s]], DB[s], SG[s])
                pltpu.async_copy(locs_hbm.at[IB[s]], LB[s], SG[s])

        pltpu.async_copy(idx_hbm.at[pl.ds(gid(0) * GR, GR)], IB[0], SI[0])

        def body(tt, carry):
            stage(2 * tt, 0)
            stage(2 * tt + 1, 1)
            return carry

        lax.fori_loop(0, (ITERS + 3) // 2, body, 0)

    outl, outd = k(idxs_flat, locs2d, data2d)
    return (outl.reshape(B, N, D), outd.reshape(B, N, C))


# speedup vs baseline: 1.0659x; 1.0010x over previous
import functools

import jax
import jax.numpy as jnp
from jax import lax
from jax.experimental import pallas as pl
from jax.experimental.pallas import tpu as pltpu
from jax.experimental.pallas import tpu_sc as plsc

_LP = 8


def kernel(idxs, locs, data):
    B, N, D = locs.shape
    C = data.shape[2]
    RT = B * N

    GR = 800
    assert RT % GR == 0
    NGROUPS = RT // GR

    info = plsc.get_sparse_core_info()
    NC, NS = info.num_cores, info.num_subcores
    NW = NC * NS
    ITERS = -(-NGROUPS // NW)

    idxs_flat = idxs.reshape(RT)
    locs2d = jnp.pad(locs.reshape(RT, D), ((0, 0), (0, _LP - D)))
    data2d = data.reshape(RT, C)

    mesh = plsc.VectorSubcoreMesh(core_axis_name="c", subcore_axis_name="s")

    @functools.partial(
        pl.kernel,
        mesh=mesh,
        out_type=[
            jax.ShapeDtypeStruct((RT, D), jnp.float32),
            jax.ShapeDtypeStruct((RT, C), jnp.float32),
        ],
        scratch_types=[
            pltpu.VMEM((GR,), jnp.int32),
            pltpu.VMEM((GR,), jnp.int32),
            pltpu.VMEM((GR, _LP), jnp.float32),
            pltpu.VMEM((GR, _LP), jnp.float32),
            pltpu.VMEM((GR, C), jnp.float32),
            pltpu.VMEM((GR, C), jnp.float32),
            pltpu.SemaphoreType.DMA,
            pltpu.SemaphoreType.DMA,
            pltpu.SemaphoreType.DMA,
            pltpu.SemaphoreType.DMA,
            pltpu.SemaphoreType.DMA,
            pltpu.SemaphoreType.DMA,
        ],
        compiler_params=pltpu.CompilerParams(use_tc_tiling_on_sc=False),
    )
    def k(idx_hbm, locs_hbm, data_hbm, outl_hbm, outd_hbm,
          ib0, ib1, lb0, lb1, db0, db1, si0, si1, sg0, sg1, sw0, sw1):
        wid = lax.axis_index("s") * NC + lax.axis_index("c")
        IB, LB, DB = (ib0, ib1), (lb0, lb1), (db0, db1)
        SI, SG, SW = (si0, si1), (sg0, sg1), (sw0, sw1)

        def gid(t):
            return wid + NW * t

        def stage(t, u):
            s, o = u, 1 - u

            @pl.when(jnp.logical_and(t >= 2, gid(t - 2) < NGROUPS))
            def _():
                g_row = gid(t - 2) * GR
                pltpu.make_async_copy(
                    DB[s], outd_hbm.at[pl.ds(g_row, GR)], SW[s]).wait()
                pltpu.make_async_copy(
                    LB[s].at[:, pl.ds(0, D)],
                    outl_hbm.at[pl.ds(g_row, GR)], SW[s]).wait()

            @pl.when(jnp.logical_and(
                jnp.logical_and(t >= 1, t <= ITERS), gid(t - 1) < NGROUPS))
            def _():
                g_row = gid(t - 1) * GR
                pltpu.make_async_copy(data_hbm.at[IB[o]], DB[o], SG[o]).wait()
                pltpu.make_async_copy(locs_hbm.at[IB[o]], LB[o], SG[o]).wait()
                pltpu.async_copy(DB[o], outd_hbm.at[pl.ds(g_row, GR)], SW[o])
                pltpu.async_copy(
                    LB[o].at[:, pl.ds(0, D)], outl_hbm.at[pl.ds(g_row, GR)], SW[o])

            @pl.when(jnp.logical_and(t + 1 <= ITERS - 1, gid(t + 1) < NGROUPS))
            def _():
                pltpu.async_copy(
                    idx_hbm.at[pl.ds(gid(t + 1) * GR, GR)], IB[o], SI[o])

            @pl.when(jnp.logical_and(t <= ITERS - 1, gid(t) < NGROUPS))
            def _():
                g_row = gid(t) * GR
                pltpu.make_async_copy(
                    idx_hbm.at[pl.ds(g_row, GR)], IB[s], SI[s]).wait()

                def conv(i, carry):
                    off = pl.multiple_of(i * 16, 16)
                    pos = g_row + off + lax.iota(jnp.int32, 16)
                    v = IB[s][pl.ds(off, 16)] + lax.div(pos, N) * N
                    pltpu.async_copy(
                        data_hbm.at[v], DB[s].at[pl.ds(off, 16)], SG[s])
                    pltpu.async_copy(
                        locs_hbm.at[v], LB[s].at[pl.ds(off, 16)], SG[s])
                    return carry

                lax.fori_loop(0, GR // 16, conv, 0)

        pltpu.async_copy(idx_hbm.at[pl.ds(gid(0) * GR, GR)], IB[0], SI[0])

        def body(tt, carry):
            stage(2 * tt, 0)
            stage(2 * tt + 1, 1)
            return carry

        lax.fori_loop(0, (ITERS + 3) // 2, body, 0)

    outl, outd = k(idxs_flat, locs2d, data2d)
    return (outl.reshape(B, N, D), outd.reshape(B, N, C))


# speedup vs baseline: 1.1265x; 1.0568x over previous
import functools

import jax
import jax.numpy as jnp
from jax import lax
from jax.experimental import pallas as pl
from jax.experimental.pallas import tpu as pltpu
from jax.experimental.pallas import tpu_sc as plsc

_LP = 8
_CP = 128


def kernel(idxs, locs, data):
    B, N, D = locs.shape
    C = data.shape[2]
    RT = B * N

    GR = 400
    assert RT % GR == 0
    NGROUPS = RT // GR

    info = plsc.get_sparse_core_info()
    NC, NS = info.num_cores, info.num_subcores
    NW = NC * NS
    ITERS = -(-NGROUPS // NW)

    idxs_flat = idxs.reshape(RT)
    locs2d = jnp.pad(locs.reshape(RT, D), ((0, 0), (0, _LP - D)))
    data2d = jnp.pad(data.reshape(RT, C), ((0, 0), (0, _CP - C)))

    mesh = plsc.VectorSubcoreMesh(core_axis_name="c", subcore_axis_name="s")

    @functools.partial(
        pl.kernel,
        mesh=mesh,
        out_type=[
            jax.ShapeDtypeStruct((RT, D), jnp.float32),
            jax.ShapeDtypeStruct((RT, _CP), jnp.float32),
        ],
        scratch_types=[
            pltpu.VMEM((GR,), jnp.int32),
            pltpu.VMEM((GR,), jnp.int32),
            pltpu.VMEM((GR, _LP), jnp.float32),
            pltpu.VMEM((GR, _LP), jnp.float32),
            pltpu.VMEM((GR, _CP), jnp.float32),
            pltpu.VMEM((GR, _CP), jnp.float32),
            pltpu.SemaphoreType.DMA,
            pltpu.SemaphoreType.DMA,
            pltpu.SemaphoreType.DMA,
            pltpu.SemaphoreType.DMA,
            pltpu.SemaphoreType.DMA,
            pltpu.SemaphoreType.DMA,
        ],
        compiler_params=pltpu.CompilerParams(use_tc_tiling_on_sc=False),
    )
    def k(idx_hbm, locs_hbm, data_hbm, outl_hbm, outd_hbm,
          ib0, ib1, lb0, lb1, db0, db1, si0, si1, sg0, sg1, sw0, sw1):
        wid = lax.axis_index("s") * NC + lax.axis_index("c")
        IB, LB, DB = (ib0, ib1), (lb0, lb1), (db0, db1)
        SI, SG, SW = (si0, si1), (sg0, sg1), (sw0, sw1)

        def gid(t):
            return wid + NW * t

        def stage(t, u):
            s, o = u, 1 - u

            @pl.when(jnp.logical_and(t >= 2, gid(t - 2) < NGROUPS))
            def _():
                g_row = gid(t - 2) * GR
                pltpu.make_async_copy(
                    DB[s], outd_hbm.at[pl.ds(g_row, GR)], SW[s]).wait()
                pltpu.make_async_copy(
                    LB[s].at[:, pl.ds(0, D)],
                    outl_hbm.at[pl.ds(g_row, GR)], SW[s]).wait()

            @pl.when(jnp.logical_and(
                jnp.logical_and(t >= 1, t <= ITERS), gid(t - 1) < NGROUPS))
            def _():
                g_row = gid(t - 1) * GR
                pltpu.make_async_copy(data_hbm.at[IB[o]], DB[o], SG[o]).wait()
                pltpu.make_async_copy(locs_hbm.at[IB[o]], LB[o], SG[o]).wait()
                pltpu.async_copy(DB[o], outd_hbm.at[pl.ds(g_row, GR)], SW[o])
                pltpu.async_copy(
                    LB[o].at[:, pl.ds(0, D)], outl_hbm.at[pl.ds(g_row, GR)], SW[o])

            @pl.when(jnp.logical_and(t + 1 <= ITERS - 1, gid(t + 1) < NGROUPS))
            def _():
                pltpu.async_copy(
                    idx_hbm.at[pl.ds(gid(t + 1) * GR, GR)], IB[o], SI[o])

            @pl.when(jnp.logical_and(t <= ITERS - 1, gid(t) < NGROUPS))
            def _():
                g_row = gid(t) * GR
                pltpu.make_async_copy(
                    idx_hbm.at[pl.ds(g_row, GR)], IB[s], SI[s]).wait()

                def conv(i, carry):
                    off = pl.multiple_of(i * 16, 16)
                    pos = g_row + off + lax.iota(jnp.int32, 16)
                    v = IB[s][pl.ds(off, 16)] + lax.div(pos, N) * N
                    pltpu.async_copy(
                        data_hbm.at[v], DB[s].at[pl.ds(off, 16)], SG[s])
                    pltpu.async_copy(
                        locs_hbm.at[v], LB[s].at[pl.ds(off, 16)], SG[s])
                    return carry

                lax.fori_loop(0, GR // 16, conv, 0)

        pltpu.async_copy(idx_hbm.at[pl.ds(gid(0) * GR, GR)], IB[0], SI[0])

        def body(tt, carry):
            stage(2 * tt, 0)
            stage(2 * tt + 1, 1)
            return carry

        lax.fori_loop(0, (ITERS + 3) // 2, body, 0)

    outl, outd = k(idxs_flat, locs2d, data2d)
    return (outl.reshape(B, N, D), outd[:, :C].reshape(B, N, C))


# speedup vs baseline: 2.4623x; 2.1858x over previous
import functools

import jax
import jax.numpy as jnp
from jax import lax
from jax.experimental import pallas as pl
from jax.experimental.pallas import tpu as pltpu
from jax.experimental.pallas import tpu_sc as plsc

_CP = 128


def kernel(idxs, locs, data):
    B, N, D = locs.shape
    C = data.shape[2]
    RT = B * N

    GR = 400
    assert RT % GR == 0
    NGROUPS = RT // GR

    info = plsc.get_sparse_core_info()
    NC, NS = info.num_cores, info.num_subcores
    NW = NC * NS
    ITERS = -(-NGROUPS // NW)

    idxs_flat = idxs.reshape(RT)
    table = jnp.pad(
        jnp.concatenate([data.reshape(RT, C), locs.reshape(RT, D)], axis=1),
        ((0, 0), (0, _CP - C - D)))

    mesh = plsc.VectorSubcoreMesh(core_axis_name="c", subcore_axis_name="s")

    @functools.partial(
        pl.kernel,
        mesh=mesh,
        out_type=jax.ShapeDtypeStruct((RT, _CP), jnp.float32),
        scratch_types=[
            pltpu.VMEM((GR,), jnp.int32),
            pltpu.VMEM((GR,), jnp.int32),
            pltpu.VMEM((GR, _CP), jnp.float32),
            pltpu.VMEM((GR, _CP), jnp.float32),
            pltpu.SemaphoreType.DMA,
            pltpu.SemaphoreType.DMA,
            pltpu.SemaphoreType.DMA,
            pltpu.SemaphoreType.DMA,
            pltpu.SemaphoreType.DMA,
            pltpu.SemaphoreType.DMA,
        ],
        compiler_params=pltpu.CompilerParams(use_tc_tiling_on_sc=False),
    )
    def k(idx_hbm, tab_hbm, out_hbm,
          ib0, ib1, db0, db1, si0, si1, sg0, sg1, sw0, sw1):
        wid = lax.axis_index("s") * NC + lax.axis_index("c")
        IB, DB = (ib0, ib1), (db0, db1)
        SI, SG, SW = (si0, si1), (sg0, sg1), (sw0, sw1)

        def gid(t):
            return wid + NW * t

        def stage(t, u):
            s, o = u, 1 - u

            @pl.when(jnp.logical_and(t >= 2, gid(t - 2) < NGROUPS))
            def _():
                pltpu.make_async_copy(
                    DB[s], out_hbm.at[pl.ds(gid(t - 2) * GR, GR)], SW[s]).wait()

            @pl.when(jnp.logical_and(
                jnp.logical_and(t >= 1, t <= ITERS), gid(t - 1) < NGROUPS))
            def _():
                pltpu.make_async_copy(tab_hbm.at[IB[o]], DB[o], SG[o]).wait()
                pltpu.async_copy(
                    DB[o], out_hbm.at[pl.ds(gid(t - 1) * GR, GR)], SW[o])

            @pl.when(jnp.logical_and(t + 1 <= ITERS - 1, gid(t + 1) < NGROUPS))
            def _():
                pltpu.async_copy(
                    idx_hbm.at[pl.ds(gid(t + 1) * GR, GR)], IB[o], SI[o])

            @pl.when(jnp.logical_and(t <= ITERS - 1, gid(t) < NGROUPS))
            def _():
                g_row = gid(t) * GR
                pltpu.make_async_copy(
                    idx_hbm.at[pl.ds(g_row, GR)], IB[s], SI[s]).wait()

                def conv(i, carry):
                    off = pl.multiple_of(i * 16, 16)
                    pos = g_row + off + lax.iota(jnp.int32, 16)
                    v = IB[s][pl.ds(off, 16)] + lax.div(pos, N) * N
                    pltpu.async_copy(
                        tab_hbm.at[v], DB[s].at[pl.ds(off, 16)], SG[s])
                    return carry

                lax.fori_loop(0, GR // 16, conv, 0)

        pltpu.async_copy(idx_hbm.at[pl.ds(gid(0) * GR, GR)], IB[0], SI[0])

        def body(tt, carry):
            stage(2 * tt, 0)
            stage(2 * tt + 1, 1)
            return carry

        lax.fori_loop(0, (ITERS + 3) // 2, body, 0)

    out = k(idxs_flat, table)
    nlocs = out[:, C:C + D].reshape(B, N, D)
    ndata = out[:, :C].reshape(B, N, C)
    return (nlocs, ndata)
